# baseline (speedup 1.0000x reference)
import functools

import jax
import jax.numpy as jnp
from jax import lax
from jax.experimental import pallas as pl
from jax.experimental.pallas import tpu as pltpu
from jax.experimental.pallas import tpu_sc as plsc

N = 10000
E = 320000
D = 128
G = 16
N_ACT = 32

NC = 2
NS = 16
NW = NC * NS
LANE = 128
CH = 80
CH0 = 160
HS = 40
TOT = NS * CH0
E_PAD = TOT * LANE
RPT = 640
N_PAD = NS * RPT

_f32 = jnp.float32
_HIGH = lax.Precision.HIGHEST


def _deg_body(dst_hbm, ones_hbm, zeros_hbm, out_hbm, dst_v, ones_v, acc):
    c = lax.axis_index("c")
    s = lax.axis_index("s")
    w = c * NS + s
    pltpu.sync_copy(dst_hbm.at[pl.ds(w * CH, CH)], dst_v)
    pltpu.sync_copy(ones_hbm, ones_v)
    pltpu.sync_copy(zeros_hbm.at[pl.ds(s * RPT, RPT)], acc.at[pl.ds(s * RPT, RPT)])
    plsc.subcore_barrier()

    def body(j, carry):
        pltpu.sync_copy(ones_v, acc.at[dst_v.at[j]], add=True)
        return carry

    lax.fori_loop(0, CH, body, 0)
    plsc.subcore_barrier()
    pltpu.sync_copy(acc.at[pl.ds(s * RPT, RPT)],
                    out_hbm.at[pl.ds(w * RPT, RPT)])


_deg_call = pl.kernel(
    _deg_body,
    out_type=jax.ShapeDtypeStruct((2 * N_PAD, D), _f32),
    mesh=plsc.VectorSubcoreMesh(core_axis_name="c", subcore_axis_name="s"),
    scratch_types=[
        pltpu.VMEM((CH, LANE), jnp.int32),
        pltpu.VMEM((LANE, D), _f32),
        pltpu.VMEM_SHARED((N_PAD, D), _f32),
    ],
)


def _agg_body(table_hbm, src_hbm, dst_hbm, zeros_hbm, out_hbm,
              src_v, dst_v, rbuf0, rbuf1, acc, sem):
    c = lax.axis_index("c")
    s = lax.axis_index("s")
    w = c * NS + s
    @pl.when(c == 0)
    def _zero():
        pltpu.sync_copy(zeros_hbm.at[pl.ds(s * RPT, RPT)],
                        acc.at[pl.ds(s * RPT, RPT)])

    plsc.subcore_barrier()

    @pl.when(c == 0)
    def _fast_core():
        for h in range(CH0 // HS):
            rb = s * CH0 + h * HS
            pltpu.sync_copy(src_hbm.at[pl.ds(rb, HS)], src_v)
            pltpu.sync_copy(dst_hbm.at[pl.ds(rb, HS)], dst_v)
            pltpu.async_copy(table_hbm.at[src_v.at[0]], rbuf0, sem)
            pltpu.async_copy(table_hbm.at[src_v.at[1]], rbuf1, sem)

            def body(g, carry):
                for b, rbuf in ((0, rbuf0), (1, rbuf1)):
                    j = 2 * g + b
                    pltpu.make_async_copy(table_hbm.at[src_v.at[j]], rbuf,
                                          sem).wait()
                    pltpu.sync_copy(rbuf, acc.at[dst_v.at[j]], add=True)

                    @pl.when(j + 2 < HS)
                    def _():
                        pltpu.async_copy(table_hbm.at[src_v.at[j + 2]], rbuf, sem)
                return carry

            lax.fori_loop(0, HS // 2, body, 0)

    plsc.subcore_barrier()

    @pl.when(c == 0)
    def _writeout():
        pltpu.sync_copy(acc.at[pl.ds(s * RPT, RPT)],
                        out_hbm.at[pl.ds(s * RPT, RPT)])


_agg_call = pl.kernel(
    _agg_body,
    out_type=jax.ShapeDtypeStruct((N_PAD, D), _f32),
    mesh=plsc.VectorSubcoreMesh(core_axis_name="c", subcore_axis_name="s"),
    scratch_types=[
        pltpu.VMEM((HS, LANE), jnp.int32),
        pltpu.VMEM((HS, LANE), jnp.int32),
        pltpu.VMEM((LANE, D), _f32),
        pltpu.VMEM((LANE, D), _f32),
        pltpu.VMEM_SHARED((N_PAD, D), _f32),
        pltpu.SemaphoreType.DMA,
    ],
)


def _tc1_body(degp_ref, x_ref, w1_ref, hs_ref, dinv_ref):
    deg = degp_ref[:N_PAD, :1] + degp_ref[N_PAD:, :1] + 1.0
    dinv = lax.rsqrt(deg)
    h = jnp.dot(x_ref[...], w1_ref[...], precision=_HIGH,
                preferred_element_type=_f32)
    hs_ref[...] = h * dinv
    dinv_ref[...] = dinv


_tc1 = pl.pallas_call(
    _tc1_body,
    out_shape=(jax.ShapeDtypeStruct((N_PAD, D), _f32),
               jax.ShapeDtypeStruct((N_PAD, 1), _f32)),
)


def _tc2_body(a_ref, hs_ref, dinv_ref, b1_ref, w2_ref, out_ref):
    dinv = dinv_ref[...]
    h1 = jnp.maximum((a_ref[...] + hs_ref[...]) * dinv + b1_ref[...], 0.0)
    out_ref[...] = jnp.dot(h1, w2_ref[...], precision=_HIGH,
                           preferred_element_type=_f32) * dinv


_tc2 = pl.pallas_call(
    _tc2_body,
    out_shape=jax.ShapeDtypeStruct((N_PAD, D), _f32),
)


def _tc3_body(a_ref, hs_ref, dinv_ref, b2_ref, batch_ref,
              wl1_ref, bl1_ref, wl2_ref, bl2_ref, q_ref):
    h2 = jnp.maximum((a_ref[...] + hs_ref[...]) * dinv_ref[...]
                     + b2_ref[...], 0.0)
    giota = lax.broadcasted_iota(jnp.int32, (1, G), 1)
    p = (batch_ref[...] == giota).astype(_f32)
    dn = (((0,), (0,)), ((), ()))
    psum = lax.dot_general(p, h2, dn, precision=_HIGH,
                           preferred_element_type=_f32)
    counts = lax.dot_general(p, jnp.ones((N_PAD, 1), _f32), dn,
                             precision=_HIGH, preferred_element_type=_f32)
    pooled = psum / jnp.maximum(counts, 1.0)
    z = jnp.maximum(jnp.dot(pooled, wl1_ref[...], precision=_HIGH,
                            preferred_element_type=_f32) + bl1_ref[...], 0.0)
    q_ref[...] = jnp.dot(z, wl2_ref[...], precision=_HIGH,
                         preferred_element_type=_f32) + bl2_ref[...]


_tc3 = pl.pallas_call(
    _tc3_body,
    out_shape=jax.ShapeDtypeStruct((G, N_ACT), _f32),
)


def kernel(x, edge_index, batch, W1, b1, W2, b2, Wl1, bl1, Wl2, bl2):
    src = edge_index[0]
    dst = edge_index[1]
    pad = E_PAD - E
    srcp = jnp.concatenate([src, jnp.zeros((pad,), jnp.int32)]
                           ).reshape(TOT, LANE)
    dstp = jnp.concatenate([dst, jnp.full((pad,), N, jnp.int32)]
                           ).reshape(TOT, LANE)
    xp = jnp.pad(x, ((0, N_PAD - N), (0, 0)))
    batchp = jnp.pad(batch, (0, N_PAD - N),
                     constant_values=-1).reshape(N_PAD, 1)
    zeros128 = jnp.zeros((N_PAD, D), _f32)
    ones128 = jnp.ones((LANE, D), _f32)

    degp = _deg_call(dstp, ones128, zeros128)
    hs1, dinv = _tc1(degp, xp, W1)
    a = _agg_call(hs1, srcp, dstp, zeros128)
    hs2 = _tc2(a, hs1, dinv, b1.reshape(1, D), W2)
    a2 = _agg_call(hs2, srcp, dstp, zeros128)
    q = _tc3(a2, hs2, dinv, b2.reshape(1, D), batchp,
             Wl1, bl1.reshape(1, D), Wl2, bl2.reshape(1, N_ACT))
    return q


# speedup vs baseline: 1.0016x; 1.0016x over previous
import functools

import jax
import jax.numpy as jnp
from jax import lax
from jax.experimental import pallas as pl
from jax.experimental.pallas import tpu as pltpu
from jax.experimental.pallas import tpu_sc as plsc

N = 10000
E = 320000
D = 128
G = 16
N_ACT = 32

NC = 2
NS = 16
NW = NC * NS
LANE = 128
CH = 80
CH0 = 160
HS = 40
TOT = NS * CH0
E_PAD = TOT * LANE
RPT = 640
N_PAD = NS * RPT

_f32 = jnp.float32
_HIGH = lax.Precision.HIGHEST


def _deg_body(dst_hbm, ones_hbm, zeros_hbm, out_hbm, dst_v, ones_v, acc):
    c = lax.axis_index("c")
    s = lax.axis_index("s")
    w = c * NS + s
    pltpu.sync_copy(dst_hbm.at[pl.ds(w * CH, CH)], dst_v)
    pltpu.sync_copy(ones_hbm, ones_v)
    pltpu.sync_copy(zeros_hbm.at[pl.ds(s * RPT, RPT)], acc.at[pl.ds(s * RPT, RPT)])
    plsc.subcore_barrier()

    def body(j, carry):
        pltpu.sync_copy(ones_v, acc.at[dst_v.at[j]], add=True)
        return carry

    lax.fori_loop(0, CH, body, 0)
    plsc.subcore_barrier()
    pltpu.sync_copy(acc.at[pl.ds(s * RPT, RPT)],
                    out_hbm.at[pl.ds(w * RPT, RPT)])


_deg_call = pl.kernel(
    _deg_body,
    out_type=jax.ShapeDtypeStruct((2 * N_PAD, D), _f32),
    mesh=plsc.VectorSubcoreMesh(core_axis_name="c", subcore_axis_name="s"),
    scratch_types=[
        pltpu.VMEM((CH, LANE), jnp.int32),
        pltpu.VMEM((LANE, D), _f32),
        pltpu.VMEM_SHARED((N_PAD, D), _f32),
    ],
)


def _agg_body(table_hbm, src_hbm, dst_hbm, zeros_hbm, out_hbm,
              src_v, dst_v, rbuf0, rbuf1, acc, sem):
    c = lax.axis_index("c")
    s = lax.axis_index("s")
    w = c * NS + s
    @pl.when(c == 0)
    def _zero():
        pltpu.sync_copy(zeros_hbm.at[pl.ds(s * RPT, RPT)],
                        acc.at[pl.ds(s * RPT, RPT)])

    plsc.subcore_barrier()

    @pl.when(c == 0)
    def _fast_core():
        def round_body(h, rcarry):
            rb = s * CH0 + h * HS
            pltpu.sync_copy(src_hbm.at[pl.ds(rb, HS)], src_v)
            pltpu.sync_copy(dst_hbm.at[pl.ds(rb, HS)], dst_v)
            pltpu.async_copy(table_hbm.at[src_v.at[0]], rbuf0, sem)
            pltpu.async_copy(table_hbm.at[src_v.at[1]], rbuf1, sem)

            def body(g, carry):
                for b, rbuf in ((0, rbuf0), (1, rbuf1)):
                    j = 2 * g + b
                    pltpu.make_async_copy(table_hbm.at[src_v.at[j]], rbuf,
                                          sem).wait()
                    pltpu.sync_copy(rbuf, acc.at[dst_v.at[j]], add=True)

                    @pl.when(j + 2 < HS)
                    def _():
                        pltpu.async_copy(table_hbm.at[src_v.at[j + 2]], rbuf, sem)
                return carry

            lax.fori_loop(0, HS // 2, body, 0)
            return rcarry

        lax.fori_loop(0, CH0 // HS, round_body, 0)

    plsc.subcore_barrier()

    @pl.when(c == 0)
    def _writeout():
        pltpu.sync_copy(acc.at[pl.ds(s * RPT, RPT)],
                        out_hbm.at[pl.ds(s * RPT, RPT)])


_agg_call = pl.kernel(
    _agg_body,
    out_type=jax.ShapeDtypeStruct((N_PAD, D), _f32),
    mesh=plsc.VectorSubcoreMesh(core_axis_name="c", subcore_axis_name="s"),
    scratch_types=[
        pltpu.VMEM((HS, LANE), jnp.int32),
        pltpu.VMEM((HS, LANE), jnp.int32),
        pltpu.VMEM((LANE, D), _f32),
        pltpu.VMEM((LANE, D), _f32),
        pltpu.VMEM_SHARED((N_PAD, D), _f32),
        pltpu.SemaphoreType.DMA,
    ],
)


def _tc1_body(degp_ref, x_ref, w1_ref, hs_ref, dinv_ref):
    deg = degp_ref[:N_PAD, :1] + degp_ref[N_PAD:, :1] + 1.0
    dinv = lax.rsqrt(deg)
    h = jnp.dot(x_ref[...], w1_ref[...], precision=_HIGH,
                preferred_element_type=_f32)
    hs_ref[...] = h * dinv
    dinv_ref[...] = dinv


_tc1 = pl.pallas_call(
    _tc1_body,
    out_shape=(jax.ShapeDtypeStruct((N_PAD, D), _f32),
               jax.ShapeDtypeStruct((N_PAD, 1), _f32)),
)


def _tc2_body(a_ref, hs_ref, dinv_ref, b1_ref, w2_ref, out_ref):
    dinv = dinv_ref[...]
    h1 = jnp.maximum((a_ref[...] + hs_ref[...]) * dinv + b1_ref[...], 0.0)
    out_ref[...] = jnp.dot(h1, w2_ref[...], precision=_HIGH,
                           preferred_element_type=_f32) * dinv


_tc2 = pl.pallas_call(
    _tc2_body,
    out_shape=jax.ShapeDtypeStruct((N_PAD, D), _f32),
)


def _tc3_body(a_ref, hs_ref, dinv_ref, b2_ref, batch_ref,
              wl1_ref, bl1_ref, wl2_ref, bl2_ref, q_ref):
    h2 = jnp.maximum((a_ref[...] + hs_ref[...]) * dinv_ref[...]
                     + b2_ref[...], 0.0)
    giota = lax.broadcasted_iota(jnp.int32, (1, G), 1)
    p = (batch_ref[...] == giota).astype(_f32)
    dn = (((0,), (0,)), ((), ()))
    psum = lax.dot_general(p, h2, dn, precision=_HIGH,
                           preferred_element_type=_f32)
    counts = lax.dot_general(p, jnp.ones((N_PAD, 1), _f32), dn,
                             precision=_HIGH, preferred_element_type=_f32)
    pooled = psum / jnp.maximum(counts, 1.0)
    z = jnp.maximum(jnp.dot(pooled, wl1_ref[...], precision=_HIGH,
                            preferred_element_type=_f32) + bl1_ref[...], 0.0)
    q_ref[...] = jnp.dot(z, wl2_ref[...], precision=_HIGH,
                         preferred_element_type=_f32) + bl2_ref[...]


_tc3 = pl.pallas_call(
    _tc3_body,
    out_shape=jax.ShapeDtypeStruct((G, N_ACT), _f32),
)


def kernel(x, edge_index, batch, W1, b1, W2, b2, Wl1, bl1, Wl2, bl2):
    src = edge_index[0]
    dst = edge_index[1]
    pad = E_PAD - E
    srcp = jnp.concatenate([src, jnp.zeros((pad,), jnp.int32)]
                           ).reshape(TOT, LANE)
    dstp = jnp.concatenate([dst, jnp.full((pad,), N, jnp.int32)]
                           ).reshape(TOT, LANE)
    xp = jnp.pad(x, ((0, N_PAD - N), (0, 0)))
    batchp = jnp.pad(batch, (0, N_PAD - N),
                     constant_values=-1).reshape(N_PAD, 1)
    zeros128 = jnp.zeros((N_PAD, D), _f32)
    ones128 = jnp.ones((LANE, D), _f32)

    degp = _deg_call(dstp, ones128, zeros128)
    hs1, dinv = _tc1(degp, xp, W1)
    a = _agg_call(hs1, srcp, dstp, zeros128)
    hs2 = _tc2(a, hs1, dinv, b1.reshape(1, D), W2)
    a2 = _agg_call(hs2, srcp, dstp, zeros128)
    q = _tc3(a2, hs2, dinv, b2.reshape(1, D), batchp,
             Wl1, bl1.reshape(1, D), Wl2, bl2.reshape(1, N_ACT))
    return q


# speedup vs baseline: 1.1482x; 1.1463x over previous
import functools

import jax
import jax.numpy as jnp
from jax import lax
from jax.experimental import pallas as pl
from jax.experimental.pallas import tpu as pltpu
from jax.experimental.pallas import tpu_sc as plsc

N = 10000
E = 320000
D = 128
G = 16
N_ACT = 32

NC = 2
NS = 16
NW = NC * NS
LANE = 128
CH = 80
CH0 = 104
CH1 = 56
TOT = NS * (CH0 + CH1)
TOT_PAD = 2624
E_PAD = TOT * LANE
RPT = 640
N_PAD = NS * RPT

_f32 = jnp.float32
_HIGH = lax.Precision.HIGHEST


def _deg_body(dst_hbm, ones_hbm, zeros_hbm, out_hbm, dst_v, ones_v, acc):
    c = lax.axis_index("c")
    s = lax.axis_index("s")
    w = c * NS + s
    pltpu.sync_copy(dst_hbm.at[pl.ds(w * CH, CH)], dst_v)
    pltpu.sync_copy(ones_hbm, ones_v)
    pltpu.sync_copy(zeros_hbm.at[pl.ds(s * RPT, RPT)], acc.at[pl.ds(s * RPT, RPT)])
    plsc.subcore_barrier()

    def body(j, carry):
        pltpu.sync_copy(ones_v, acc.at[dst_v.at[j]], add=True)
        return carry

    lax.fori_loop(0, CH, body, 0)
    plsc.subcore_barrier()
    pltpu.sync_copy(acc.at[pl.ds(s * RPT, RPT)],
                    out_hbm.at[pl.ds(w * RPT, RPT)])


_deg_call = pl.kernel(
    _deg_body,
    out_type=jax.ShapeDtypeStruct((2 * N_PAD, D), _f32),
    mesh=plsc.VectorSubcoreMesh(core_axis_name="c", subcore_axis_name="s"),
    scratch_types=[
        pltpu.VMEM((CH, LANE), jnp.int32),
        pltpu.VMEM((LANE, D), _f32),
        pltpu.VMEM_SHARED((N_PAD, D), _f32),
    ],
)


def _agg_body(table_hbm, src_hbm, dst_hbm, zeros_hbm, out_hbm,
              src_v, dst_v, rbuf, acc, sem):
    c = lax.axis_index("c")
    s = lax.axis_index("s")
    w = c * NS + s
    chc = CH0 + c * (CH1 - CH0)
    rb = c * (NS * CH0) + s * chc
    pltpu.sync_copy(zeros_hbm.at[pl.ds(s * RPT, RPT)], acc.at[pl.ds(s * RPT, RPT)])
    pltpu.sync_copy(src_hbm.at[pl.ds(rb, CH0)], src_v)
    pltpu.sync_copy(dst_hbm.at[pl.ds(rb, CH0)], dst_v)
    plsc.subcore_barrier()

    def body(j, carry):
        pltpu.async_copy(table_hbm.at[src_v.at[j]], rbuf, sem).wait()
        pltpu.sync_copy(rbuf, acc.at[dst_v.at[j]], add=True)
        return carry

    lax.fori_loop(0, chc, body, 0)
    plsc.subcore_barrier()
    pltpu.sync_copy(acc.at[pl.ds(s * RPT, RPT)],
                    out_hbm.at[pl.ds(w * RPT, RPT)])


_agg_call = pl.kernel(
    _agg_body,
    out_type=jax.ShapeDtypeStruct((2 * N_PAD, D), _f32),
    mesh=plsc.VectorSubcoreMesh(core_axis_name="c", subcore_axis_name="s"),
    scratch_types=[
        pltpu.VMEM((CH0, LANE), jnp.int32),
        pltpu.VMEM((CH0, LANE), jnp.int32),
        pltpu.VMEM((LANE, D), _f32),
        pltpu.VMEM_SHARED((N_PAD, D), _f32),
        pltpu.SemaphoreType.DMA,
    ],
)


def _tc1_body(degp_ref, x_ref, w1_ref, hs_ref, dinv_ref):
    deg = degp_ref[:N_PAD, :1] + degp_ref[N_PAD:, :1] + 1.0
    dinv = lax.rsqrt(deg)
    h = jnp.dot(x_ref[...], w1_ref[...], precision=_HIGH,
                preferred_element_type=_f32)
    hs_ref[...] = h * dinv
    dinv_ref[...] = dinv


_tc1 = pl.pallas_call(
    _tc1_body,
    out_shape=(jax.ShapeDtypeStruct((N_PAD, D), _f32),
               jax.ShapeDtypeStruct((N_PAD, 1), _f32)),
)


def _tc2_body(a_ref, hs_ref, dinv_ref, b1_ref, w2_ref, out_ref):
    dinv = dinv_ref[...]
    h1 = jnp.maximum((a_ref[:N_PAD] + a_ref[N_PAD:] + hs_ref[...]) * dinv
                     + b1_ref[...], 0.0)
    out_ref[...] = jnp.dot(h1, w2_ref[...], precision=_HIGH,
                           preferred_element_type=_f32) * dinv


_tc2 = pl.pallas_call(
    _tc2_body,
    out_shape=jax.ShapeDtypeStruct((N_PAD, D), _f32),
)


def _tc3_body(a_ref, hs_ref, dinv_ref, b2_ref, batch_ref,
              wl1_ref, bl1_ref, wl2_ref, bl2_ref, q_ref):
    h2 = jnp.maximum((a_ref[:N_PAD] + a_ref[N_PAD:] + hs_ref[...]) * dinv_ref[...]
                     + b2_ref[...], 0.0)
    giota = lax.broadcasted_iota(jnp.int32, (1, G), 1)
    p = (batch_ref[...] == giota).astype(_f32)
    dn = (((0,), (0,)), ((), ()))
    psum = lax.dot_general(p, h2, dn, precision=_HIGH,
                           preferred_element_type=_f32)
    counts = lax.dot_general(p, jnp.ones((N_PAD, 1), _f32), dn,
                             precision=_HIGH, preferred_element_type=_f32)
    pooled = psum / jnp.maximum(counts, 1.0)
    z = jnp.maximum(jnp.dot(pooled, wl1_ref[...], precision=_HIGH,
                            preferred_element_type=_f32) + bl1_ref[...], 0.0)
    q_ref[...] = jnp.dot(z, wl2_ref[...], precision=_HIGH,
                         preferred_element_type=_f32) + bl2_ref[...]


_tc3 = pl.pallas_call(
    _tc3_body,
    out_shape=jax.ShapeDtypeStruct((G, N_ACT), _f32),
)


def kernel(x, edge_index, batch, W1, b1, W2, b2, Wl1, bl1, Wl2, bl2):
    src = edge_index[0]
    dst = edge_index[1]
    pad = E_PAD - E
    srcp = jnp.concatenate([src, jnp.zeros((pad,), jnp.int32)]
                           ).reshape(TOT, LANE)
    dstp = jnp.concatenate([dst, jnp.full((pad,), N, jnp.int32)]
                           ).reshape(TOT, LANE)
    srcp = jnp.pad(srcp, ((0, TOT_PAD - TOT), (0, 0)))
    dstp = jnp.pad(dstp, ((0, TOT_PAD - TOT), (0, 0)), constant_values=N)
    xp = jnp.pad(x, ((0, N_PAD - N), (0, 0)))
    batchp = jnp.pad(batch, (0, N_PAD - N),
                     constant_values=-1).reshape(N_PAD, 1)
    zeros128 = jnp.zeros((N_PAD, D), _f32)
    ones128 = jnp.ones((LANE, D), _f32)

    degp = _deg_call(dstp, ones128, zeros128)
    hs1, dinv = _tc1(degp, xp, W1)
    a = _agg_call(hs1, srcp, dstp, zeros128)
    hs2 = _tc2(a, hs1, dinv, b1.reshape(1, D), W2)
    a2 = _agg_call(hs2, srcp, dstp, zeros128)
    q = _tc3(a2, hs2, dinv, b2.reshape(1, D), batchp,
             Wl1, bl1.reshape(1, D), Wl2, bl2.reshape(1, N_ACT))
    return q


# speedup vs baseline: 2.1297x; 1.8549x over previous
import functools

import jax
import jax.numpy as jnp
from jax import lax
from jax.experimental import pallas as pl
from jax.experimental.pallas import tpu as pltpu
from jax.experimental.pallas import tpu_sc as plsc

N = 10000
E = 320000
D = 128
G = 16
N_ACT = 32

NC = 2
NS = 16
NW = NC * NS
LANE = 128
CH = 80
CH0 = 104
CH1 = 56
TOT = NS * (CH0 + CH1)
TOT_PAD = 2624
E_PAD = TOT * LANE
RPT = 640
N_PAD = NS * RPT

_f32 = jnp.float32
_HIGH = lax.Precision.HIGHEST


def _deg_body(dst_hbm, ones_hbm, zeros_hbm, out_hbm, dst_v, ones_v, acc):
    c = lax.axis_index("c")
    s = lax.axis_index("s")
    w = c * NS + s
    pltpu.sync_copy(dst_hbm.at[pl.ds(w * CH, CH)], dst_v)
    pltpu.sync_copy(ones_hbm, ones_v)
    pltpu.sync_copy(zeros_hbm.at[pl.ds(s * RPT, RPT)], acc.at[pl.ds(s * RPT, RPT)])
    plsc.subcore_barrier()

    def body(j, carry):
        pltpu.sync_copy(ones_v, acc.at[dst_v.at[j]], add=True)
        return carry

    lax.fori_loop(0, CH, body, 0)
    plsc.subcore_barrier()
    pltpu.sync_copy(acc.at[pl.ds(s * RPT, RPT)],
                    out_hbm.at[pl.ds(w * RPT, RPT)])


_deg_call = pl.kernel(
    _deg_body,
    out_type=jax.ShapeDtypeStruct((2 * N_PAD, D), _f32),
    mesh=plsc.VectorSubcoreMesh(core_axis_name="c", subcore_axis_name="s"),
    scratch_types=[
        pltpu.VMEM((CH, LANE), jnp.int32),
        pltpu.VMEM((LANE, D), _f32),
        pltpu.VMEM_SHARED((N_PAD, D), _f32),
    ],
)


def _agg_body(table_hbm, src_hbm, dst_hbm, zeros_hbm, out_hbm,
              src_v, dst_v, rbuf, acc, sem):
    c = lax.axis_index("c")
    s = lax.axis_index("s")
    w = c * NS + s
    chc = CH0 + c * (CH1 - CH0)
    rb = c * (NS * CH0) + s * chc
    pltpu.sync_copy(zeros_hbm.at[pl.ds(s * RPT, RPT)], acc.at[pl.ds(s * RPT, RPT)])
    pltpu.sync_copy(src_hbm.at[pl.ds(rb, CH0)], src_v)
    pltpu.sync_copy(dst_hbm.at[pl.ds(rb, CH0)], dst_v)
    plsc.subcore_barrier()

    def body(j, carry):
        pltpu.async_copy(table_hbm.at[src_v.at[j]], rbuf, sem).wait()
        pltpu.sync_copy(rbuf, acc.at[dst_v.at[j]], add=True)
        return carry

    lax.fori_loop(0, chc, body, 0)
    plsc.subcore_barrier()
    pltpu.sync_copy(acc.at[pl.ds(s * RPT, RPT)],
                    out_hbm.at[pl.ds(w * RPT, RPT)])


_agg_call = pl.kernel(
    _agg_body,
    out_type=jax.ShapeDtypeStruct((2 * N_PAD, D), _f32),
    mesh=plsc.VectorSubcoreMesh(core_axis_name="c", subcore_axis_name="s"),
    scratch_types=[
        pltpu.VMEM((CH0, LANE), jnp.int32),
        pltpu.VMEM((CH0, LANE), jnp.int32),
        pltpu.VMEM((LANE, D), _f32),
        pltpu.VMEM_SHARED((N_PAD, D), _f32),
        pltpu.SemaphoreType.DMA,
    ],
)


def _tc1_body(degp_ref, x_ref, w1_ref, hs_ref, dinv_ref):
    deg = degp_ref[:N_PAD, :1] + degp_ref[N_PAD:, :1] + 1.0
    dinv = lax.rsqrt(deg)
    h = jnp.dot(x_ref[...], w1_ref[...], precision=_HIGH,
                preferred_element_type=_f32)
    hs_ref[...] = h * dinv
    dinv_ref[...] = dinv


_tc1 = pl.pallas_call(
    _tc1_body,
    out_shape=(jax.ShapeDtypeStruct((N_PAD, D), _f32),
               jax.ShapeDtypeStruct((N_PAD, 1), _f32)),
)


def _tc2_body(a_ref, hs_ref, dinv_ref, b1_ref, w2_ref, out_ref):
    dinv = dinv_ref[...]
    h1 = jnp.maximum((a_ref[:N_PAD] + a_ref[N_PAD:] + hs_ref[...]) * dinv
                     + b1_ref[...], 0.0)
    out_ref[...] = jnp.dot(h1, w2_ref[...], precision=_HIGH,
                           preferred_element_type=_f32) * dinv


_tc2 = pl.pallas_call(
    _tc2_body,
    out_shape=jax.ShapeDtypeStruct((N_PAD, D), _f32),
)


def _tc3_body(a_ref, hs_ref, dinv_ref, b2_ref, batch_ref,
              wl1_ref, bl1_ref, wl2_ref, bl2_ref, q_ref):
    h2 = jnp.maximum((a_ref[:N_PAD] + a_ref[N_PAD:] + hs_ref[...]) * dinv_ref[...]
                     + b2_ref[...], 0.0)
    giota = lax.broadcasted_iota(jnp.int32, (1, G), 1)
    p = (batch_ref[...] == giota).astype(_f32)
    dn = (((0,), (0,)), ((), ()))
    psum = lax.dot_general(p, h2, dn, precision=_HIGH,
                           preferred_element_type=_f32)
    counts = lax.dot_general(p, jnp.ones((N_PAD, 1), _f32), dn,
                             precision=_HIGH, preferred_element_type=_f32)
    pooled = psum / jnp.maximum(counts, 1.0)
    z = jnp.maximum(jnp.dot(pooled, wl1_ref[...], precision=_HIGH,
                            preferred_element_type=_f32) + bl1_ref[...], 0.0)
    q_ref[...] = jnp.dot(z, wl2_ref[...], precision=_HIGH,
                         preferred_element_type=_f32) + bl2_ref[...]


_tc3 = pl.pallas_call(
    _tc3_body,
    out_shape=jax.ShapeDtypeStruct((G, N_ACT), _f32),
)


def kernel(x, edge_index, batch, W1, b1, W2, b2, Wl1, bl1, Wl2, bl2):
    src = edge_index[0]
    dst = edge_index[1]
    pad = E_PAD - E
    pad_src = (jnp.arange(pad, dtype=jnp.int32) * 53) % N
    pad_dst = N + (jnp.arange(pad, dtype=jnp.int32) % (N_PAD - N))
    srcp = jnp.concatenate([src, pad_src]).reshape(TOT, LANE)
    dstp = jnp.concatenate([dst, pad_dst]).reshape(TOT, LANE)
    srcp = jnp.pad(srcp, ((0, TOT_PAD - TOT), (0, 0)))
    dstp = jnp.pad(dstp, ((0, TOT_PAD - TOT), (0, 0)), constant_values=N)
    xp = jnp.pad(x, ((0, N_PAD - N), (0, 0)))
    batchp = jnp.pad(batch, (0, N_PAD - N),
                     constant_values=-1).reshape(N_PAD, 1)
    zeros128 = jnp.zeros((N_PAD, D), _f32)
    ones128 = jnp.ones((LANE, D), _f32)

    degp = _deg_call(dstp, ones128, zeros128)
    hs1, dinv = _tc1(degp, xp, W1)
    a = _agg_call(hs1, srcp, dstp, zeros128)
    hs2 = _tc2(a, hs1, dinv, b1.reshape(1, D), W2)
    a2 = _agg_call(hs2, srcp, dstp, zeros128)
    q = _tc3(a2, hs2, dinv, b2.reshape(1, D), batchp,
             Wl1, bl1.reshape(1, D), Wl2, bl2.reshape(1, N_ACT))
    return q


# speedup vs baseline: 3.4526x; 1.6211x over previous
import functools

import jax
import jax.numpy as jnp
from jax import lax
from jax.experimental import pallas as pl
from jax.experimental.pallas import tpu as pltpu
from jax.experimental.pallas import tpu_sc as plsc

N = 10000
E = 320000
D = 128
G = 16
N_ACT = 32

NC = 2
NS = 16
NW = NC * NS
LANE = 128
CH = 80
CHA = 80
HS = 40
TOT = NW * CHA
E_PAD = TOT * LANE
RPT = 640
N_PAD = NS * RPT

_f32 = jnp.float32
_HIGH = lax.Precision.HIGHEST


def _deg_body(dst_hbm, ones_hbm, zeros_hbm, out_hbm, dst_v, ones_v, acc):
    c = lax.axis_index("c")
    s = lax.axis_index("s")
    w = c * NS + s
    pltpu.sync_copy(dst_hbm.at[pl.ds(w * CH, CH)], dst_v)
    pltpu.sync_copy(ones_hbm, ones_v)
    pltpu.sync_copy(zeros_hbm.at[pl.ds(s * RPT, RPT)], acc.at[pl.ds(s * RPT, RPT)])
    plsc.subcore_barrier()

    def body(j, carry):
        pltpu.sync_copy(ones_v, acc.at[dst_v.at[j]], add=True)
        return carry

    lax.fori_loop(0, CH, body, 0)
    plsc.subcore_barrier()
    pltpu.sync_copy(acc.at[pl.ds(s * RPT, RPT)],
                    out_hbm.at[pl.ds(w * RPT, RPT)])


_deg_call = pl.kernel(
    _deg_body,
    out_type=jax.ShapeDtypeStruct((2 * N_PAD, D), _f32),
    mesh=plsc.VectorSubcoreMesh(core_axis_name="c", subcore_axis_name="s"),
    scratch_types=[
        pltpu.VMEM((CH, LANE), jnp.int32),
        pltpu.VMEM((LANE, D), _f32),
        pltpu.VMEM_SHARED((N_PAD, D), _f32),
    ],
)


def _agg_body(table_hbm, src_hbm, dst_hbm, zeros_hbm, out_hbm,
              src_v, dst_v, rbuf0, rbuf1, acc, sem):
    c = lax.axis_index("c")
    s = lax.axis_index("s")
    w = c * NS + s
    pltpu.sync_copy(zeros_hbm.at[pl.ds(s * RPT, RPT)], acc.at[pl.ds(s * RPT, RPT)])
    plsc.subcore_barrier()

    def round_body(h, rcarry):
        rb = w * CHA + h * HS
        pltpu.sync_copy(src_hbm.at[pl.ds(rb, HS)], src_v)
        pltpu.sync_copy(dst_hbm.at[pl.ds(rb, HS)], dst_v)
        pltpu.async_copy(table_hbm.at[src_v.at[0]], rbuf0, sem)
        pltpu.async_copy(table_hbm.at[src_v.at[1]], rbuf1, sem)

        def body(g, carry):
            for b, rbuf in ((0, rbuf0), (1, rbuf1)):
                j = 2 * g + b
                pltpu.make_async_copy(table_hbm.at[src_v.at[j]], rbuf,
                                      sem).wait()
                pltpu.sync_copy(rbuf, acc.at[dst_v.at[j]], add=True)

                @pl.when(j + 2 < HS)
                def _():
                    pltpu.async_copy(table_hbm.at[src_v.at[j + 2]], rbuf, sem)
            return carry

        lax.fori_loop(0, HS // 2, body, 0)
        return rcarry

    lax.fori_loop(0, CHA // HS, round_body, 0)
    plsc.subcore_barrier()
    pltpu.sync_copy(acc.at[pl.ds(s * RPT, RPT)],
                    out_hbm.at[pl.ds(w * RPT, RPT)])


_agg_call = pl.kernel(
    _agg_body,
    out_type=jax.ShapeDtypeStruct((2 * N_PAD, D), _f32),
    mesh=plsc.VectorSubcoreMesh(core_axis_name="c", subcore_axis_name="s"),
    scratch_types=[
        pltpu.VMEM((HS, LANE), jnp.int32),
        pltpu.VMEM((HS, LANE), jnp.int32),
        pltpu.VMEM((LANE, D), _f32),
        pltpu.VMEM((LANE, D), _f32),
        pltpu.VMEM_SHARED((N_PAD, D), _f32),
        pltpu.SemaphoreType.DMA,
    ],
)


def _tc1_body(degp_ref, x_ref, w1_ref, hs_ref, dinv_ref):
    deg = degp_ref[:N_PAD, :1] + degp_ref[N_PAD:, :1] + 1.0
    dinv = lax.rsqrt(deg)
    h = jnp.dot(x_ref[...], w1_ref[...], precision=_HIGH,
                preferred_element_type=_f32)
    hs_ref[...] = h * dinv
    dinv_ref[...] = dinv


_tc1 = pl.pallas_call(
    _tc1_body,
    out_shape=(jax.ShapeDtypeStruct((N_PAD, D), _f32),
               jax.ShapeDtypeStruct((N_PAD, 1), _f32)),
)


def _tc2_body(a_ref, hs_ref, dinv_ref, b1_ref, w2_ref, out_ref):
    dinv = dinv_ref[...]
    h1 = jnp.maximum((a_ref[:N_PAD] + a_ref[N_PAD:] + hs_ref[...]) * dinv
                     + b1_ref[...], 0.0)
    out_ref[...] = jnp.dot(h1, w2_ref[...], precision=_HIGH,
                           preferred_element_type=_f32) * dinv


_tc2 = pl.pallas_call(
    _tc2_body,
    out_shape=jax.ShapeDtypeStruct((N_PAD, D), _f32),
)


def _tc3_body(a_ref, hs_ref, dinv_ref, b2_ref, batch_ref,
              wl1_ref, bl1_ref, wl2_ref, bl2_ref, q_ref):
    h2 = jnp.maximum((a_ref[:N_PAD] + a_ref[N_PAD:] + hs_ref[...]) * dinv_ref[...]
                     + b2_ref[...], 0.0)
    giota = lax.broadcasted_iota(jnp.int32, (1, G), 1)
    p = (batch_ref[...] == giota).astype(_f32)
    dn = (((0,), (0,)), ((), ()))
    psum = lax.dot_general(p, h2, dn, precision=_HIGH,
                           preferred_element_type=_f32)
    counts = lax.dot_general(p, jnp.ones((N_PAD, 1), _f32), dn,
                             precision=_HIGH, preferred_element_type=_f32)
    pooled = psum / jnp.maximum(counts, 1.0)
    z = jnp.maximum(jnp.dot(pooled, wl1_ref[...], precision=_HIGH,
                            preferred_element_type=_f32) + bl1_ref[...], 0.0)
    q_ref[...] = jnp.dot(z, wl2_ref[...], precision=_HIGH,
                         preferred_element_type=_f32) + bl2_ref[...]


_tc3 = pl.pallas_call(
    _tc3_body,
    out_shape=jax.ShapeDtypeStruct((G, N_ACT), _f32),
)


def kernel(x, edge_index, batch, W1, b1, W2, b2, Wl1, bl1, Wl2, bl2):
    src = edge_index[0]
    dst = edge_index[1]
    pad = E_PAD - E
    pad_src = (jnp.arange(pad, dtype=jnp.int32) * 53) % N
    pad_dst = N + (jnp.arange(pad, dtype=jnp.int32) % (N_PAD - N))
    srcp = jnp.concatenate([src, pad_src]).reshape(TOT, LANE)
    dstp = jnp.concatenate([dst, pad_dst]).reshape(TOT, LANE)
    xp = jnp.pad(x, ((0, N_PAD - N), (0, 0)))
    batchp = jnp.pad(batch, (0, N_PAD - N),
                     constant_values=-1).reshape(N_PAD, 1)
    zeros128 = jnp.zeros((N_PAD, D), _f32)
    ones128 = jnp.ones((LANE, D), _f32)

    degp = _deg_call(dstp, ones128, zeros128)
    hs1, dinv = _tc1(degp, xp, W1)
    a = _agg_call(hs1, srcp, dstp, zeros128)
    hs2 = _tc2(a, hs1, dinv, b1.reshape(1, D), W2)
    a2 = _agg_call(hs2, srcp, dstp, zeros128)
    q = _tc3(a2, hs2, dinv, b2.reshape(1, D), batchp,
             Wl1, bl1.reshape(1, D), Wl2, bl2.reshape(1, N_ACT))
    return q


# speedup vs baseline: 3.4589x; 1.0019x over previous
import functools

import jax
import jax.numpy as jnp
from jax import lax
from jax.experimental import pallas as pl
from jax.experimental.pallas import tpu as pltpu
from jax.experimental.pallas import tpu_sc as plsc

N = 10000
E = 320000
D = 128
G = 16
N_ACT = 32

NC = 2
NS = 16
NW = NC * NS
LANE = 128
CH = 80
CHA = 80
HS = 40
TOT = NW * CHA
E_PAD = TOT * LANE
RPT = 640
N_PAD = NS * RPT

_f32 = jnp.float32
_HIGH = lax.Precision.HIGHEST


def _deg_body(dst_hbm, ones_hbm, zeros_hbm, out_hbm, dst_v, ones_v, acc):
    c = lax.axis_index("c")
    s = lax.axis_index("s")
    w = c * NS + s
    pltpu.sync_copy(dst_hbm.at[pl.ds(w * CH, CH)], dst_v)
    pltpu.sync_copy(ones_hbm, ones_v)
    pltpu.sync_copy(zeros_hbm.at[pl.ds(s * RPT, RPT)], acc.at[pl.ds(s * RPT, RPT)])
    plsc.subcore_barrier()

    def body(j, carry):
        pltpu.sync_copy(ones_v, acc.at[dst_v.at[j]], add=True)
        return carry

    lax.fori_loop(0, CH, body, 0)
    plsc.subcore_barrier()
    pltpu.sync_copy(acc.at[pl.ds(s * RPT, RPT)],
                    out_hbm.at[pl.ds(w * RPT, RPT)])


_deg_call = pl.kernel(
    _deg_body,
    out_type=jax.ShapeDtypeStruct((2 * N_PAD, D), _f32),
    mesh=plsc.VectorSubcoreMesh(core_axis_name="c", subcore_axis_name="s"),
    scratch_types=[
        pltpu.VMEM((CH, LANE), jnp.int32),
        pltpu.VMEM((LANE, D), _f32),
        pltpu.VMEM_SHARED((N_PAD, D), _f32),
    ],
)


def _agg_body(table_hbm, src_hbm, dst_hbm, zeros_hbm, out_hbm,
              src_v, dst_v, rbuf0, rbuf1, acc, sem):
    c = lax.axis_index("c")
    s = lax.axis_index("s")
    w = c * NS + s
    pltpu.sync_copy(zeros_hbm.at[pl.ds(s * RPT, RPT)], acc.at[pl.ds(s * RPT, RPT)])
    plsc.subcore_barrier()

    def round_body(h, rcarry):
        rb = w * CHA + h * HS
        pltpu.sync_copy(src_hbm.at[pl.ds(rb, HS)], src_v)
        pltpu.sync_copy(dst_hbm.at[pl.ds(rb, HS)], dst_v)
        pltpu.async_copy(table_hbm.at[src_v.at[0]], rbuf0, sem)
        pltpu.async_copy(table_hbm.at[src_v.at[1]], rbuf1, sem)

        def body(g, carry):
            for b, rbuf in ((0, rbuf0), (1, rbuf1)):
                j = 2 * g + b
                pltpu.make_async_copy(table_hbm.at[src_v.at[j]], rbuf,
                                      sem).wait()
                pltpu.sync_copy(rbuf, acc.at[dst_v.at[j]], add=True)

                @pl.when(j + 2 < HS)
                def _():
                    pltpu.async_copy(table_hbm.at[src_v.at[j + 2]], rbuf, sem)
            return carry

        lax.fori_loop(0, HS // 2, body, 0)
        return rcarry

    lax.fori_loop(0, CHA // HS, round_body, 0)
    plsc.subcore_barrier()
    pltpu.sync_copy(acc.at[pl.ds(s * RPT, RPT)],
                    out_hbm.at[pl.ds(w * RPT, RPT)])


_agg_call = pl.kernel(
    _agg_body,
    out_type=jax.ShapeDtypeStruct((2 * N_PAD, D), _f32),
    mesh=plsc.VectorSubcoreMesh(core_axis_name="c", subcore_axis_name="s"),
    scratch_types=[
        pltpu.VMEM((HS, LANE), jnp.int32),
        pltpu.VMEM((HS, LANE), jnp.int32),
        pltpu.VMEM((LANE, D), _f32),
        pltpu.VMEM((LANE, D), _f32),
        pltpu.VMEM_SHARED((N_PAD, D), _f32),
        pltpu.SemaphoreType.DMA,
    ],
)


def _tc1a_body(x_ref, w1_ref, u_ref):
    u_ref[...] = jnp.dot(x_ref[...], w1_ref[...], precision=_HIGH,
                         preferred_element_type=_f32)


_tc1a = pl.pallas_call(
    _tc1a_body,
    out_shape=jax.ShapeDtypeStruct((N_PAD, D), _f32),
)


def _tc1b_body(degp_ref, u_ref, hs_ref, dinv_ref):
    deg = degp_ref[:N_PAD, :1] + degp_ref[N_PAD:, :1] + 1.0
    dinv = lax.rsqrt(deg)
    hs_ref[...] = u_ref[...] * dinv
    dinv_ref[...] = dinv


_tc1b = pl.pallas_call(
    _tc1b_body,
    out_shape=(jax.ShapeDtypeStruct((N_PAD, D), _f32),
               jax.ShapeDtypeStruct((N_PAD, 1), _f32)),
)


def _tc2_body(a_ref, hs_ref, dinv_ref, b1_ref, w2_ref, out_ref):
    dinv = dinv_ref[...]
    h1 = jnp.maximum((a_ref[:N_PAD] + a_ref[N_PAD:] + hs_ref[...]) * dinv
                     + b1_ref[...], 0.0)
    out_ref[...] = jnp.dot(h1, w2_ref[...], precision=_HIGH,
                           preferred_element_type=_f32) * dinv


_tc2 = pl.pallas_call(
    _tc2_body,
    out_shape=jax.ShapeDtypeStruct((N_PAD, D), _f32),
)


def _tc3_body(a_ref, hs_ref, dinv_ref, b2_ref, batch_ref,
              wl1_ref, bl1_ref, wl2_ref, bl2_ref, q_ref):
    h2 = jnp.maximum((a_ref[:N_PAD] + a_ref[N_PAD:] + hs_ref[...]) * dinv_ref[...]
                     + b2_ref[...], 0.0)
    giota = lax.broadcasted_iota(jnp.int32, (1, G), 1)
    p = (batch_ref[...] == giota).astype(_f32)
    dn = (((0,), (0,)), ((), ()))
    psum = lax.dot_general(p, h2, dn, precision=_HIGH,
                           preferred_element_type=_f32)
    counts = lax.dot_general(p, jnp.ones((N_PAD, 1), _f32), dn,
                             precision=_HIGH, preferred_element_type=_f32)
    pooled = psum / jnp.maximum(counts, 1.0)
    z = jnp.maximum(jnp.dot(pooled, wl1_ref[...], precision=_HIGH,
                            preferred_element_type=_f32) + bl1_ref[...], 0.0)
    q_ref[...] = jnp.dot(z, wl2_ref[...], precision=_HIGH,
                         preferred_element_type=_f32) + bl2_ref[...]


_tc3 = pl.pallas_call(
    _tc3_body,
    out_shape=jax.ShapeDtypeStruct((G, N_ACT), _f32),
)


def kernel(x, edge_index, batch, W1, b1, W2, b2, Wl1, bl1, Wl2, bl2):
    src = edge_index[0]
    dst = edge_index[1]
    pad = E_PAD - E
    pad_src = (jnp.arange(pad, dtype=jnp.int32) * 53) % N
    pad_dst = N + (jnp.arange(pad, dtype=jnp.int32) % (N_PAD - N))
    srcp = jnp.concatenate([src, pad_src]).reshape(TOT, LANE)
    dstp = jnp.concatenate([dst, pad_dst]).reshape(TOT, LANE)
    xp = jnp.pad(x, ((0, N_PAD - N), (0, 0)))
    batchp = jnp.pad(batch, (0, N_PAD - N),
                     constant_values=-1).reshape(N_PAD, 1)
    zeros128 = jnp.zeros((N_PAD, D), _f32)
    ones128 = jnp.ones((LANE, D), _f32)

    degp = _deg_call(dstp, ones128, zeros128)
    u1 = _tc1a(xp, W1)
    hs1, dinv = _tc1b(degp, u1)
    a = _agg_call(hs1, srcp, dstp, zeros128)
    hs2 = _tc2(a, hs1, dinv, b1.reshape(1, D), W2)
    a2 = _agg_call(hs2, srcp, dstp, zeros128)
    q = _tc3(a2, hs2, dinv, b2.reshape(1, D), batchp,
             Wl1, bl1.reshape(1, D), Wl2, bl2.reshape(1, N_ACT))
    return q
